# baseline (speedup 1.0000x reference)
import jax
import jax.numpy as jnp
from jax import lax
from jax.experimental import pallas as pl
from jax.experimental.pallas import tpu as pltpu
from jax.experimental.pallas import tpu_sc as plsc

B = 8
N = 512
H = 16
S = 512
L = 16
NC = 2
NS = 16
NW = NC * NS
BK = 128
PT = 136
NB = N // BK
TOT = B * NB * NB
PER = TOT // NW
ISUB = 8
JV = BK // L


def _body(ab_hbm, sp_hbm, wt_hbm, wrt_hbm, out_hbm,
          spA, spB, spBT, abA, wv, wrv, outv0, outv1, sem0, sem1):
    c = lax.axis_index("c")
    s = lax.axis_index("s")
    wid = s * NC + c
    pltpu.sync_copy(wt_hbm, wv)
    pltpu.sync_copy(wrt_hbm, wrv)
    lane = lax.iota(jnp.int32, L)
    lane_pt = lane * PT

    dummy = out_hbm.at[0, :, pl.ds(0, ISUB), pl.ds(0, BK)]
    pltpu.async_copy(dummy, outv0, sem0)
    pltpu.async_copy(dummy, outv1, sem1)

    def fill(outv, isub):
        @plsc.parallel_loop(0, ISUB * JV, unroll=1)
        def pix_body(p):
            i2 = p // JV
            jv = p % JV
            i = isub * ISUB + i2
            v_idx = spA[i, pl.ds(jv * L, L)]
            vt_idx = spBT[pl.ds(i * PT + jv * L, L)]
            ab_v = abA[i, pl.ds(jv * L, L)]
            for hp in range(H // 2):
                g = plsc.load_gather(wv.at[pl.ds(hp * S, S)], [v_idx])
                gr = plsc.load_gather(wrv.at[pl.ds(hp * S, S)], [vt_idx])
                ssum = (plsc.bitcast(g, jnp.bfloat16)
                        + plsc.bitcast(gr, jnp.bfloat16))
                lo, hi = plsc.unpack(ssum, format=plsc.PackFormat.INTERLEAVED,
                                     preferred_element_type=jnp.float32)
                outv[2 * hp, i2, pl.ds(jv * L, L)] = lo + ab_v
                outv[2 * hp + 1, i2, pl.ds(jv * L, L)] = hi + ab_v

    def block_body(k, carry):
        t = wid * PER + k
        b = t // (NB * NB)
        r = t % (NB * NB)
        i0 = (r // NB) * BK
        j0 = (r % NB) * BK
        pltpu.sync_copy(sp_hbm.at[b, pl.ds(i0, BK), pl.ds(j0, BK)], spA)
        pltpu.sync_copy(sp_hbm.at[b, pl.ds(j0, BK), pl.ds(i0, BK)], spB)
        pltpu.sync_copy(ab_hbm.at[b, pl.ds(i0, BK), pl.ds(j0, BK)], abA)

        @plsc.parallel_loop(0, BK * JV, unroll=1)
        def tr_body(g):
            j = g // JV
            iv = g % JV
            row = spB[j, pl.ds(iv * L, L)]
            plsc.store_scatter(spBT, [lane_pt + (iv * L * PT + j)], row)

        def isub2_body(k2, _):
            for half, (ov, sem) in enumerate(((outv0, sem0), (outv1, sem1))):
                isub = k2 * 2 + half
                dst = out_hbm.at[b, :, pl.ds(i0 + isub * ISUB, ISUB),
                                 pl.ds(j0, BK)]
                pltpu.make_async_copy(ov, dst, sem).wait()
                fill(ov, isub)
                pltpu.async_copy(ov, dst, sem)
            return _

        lax.fori_loop(0, (BK // ISUB) // 2, isub2_body, 0)
        return carry

    lax.fori_loop(0, PER, block_body, 0)
    pltpu.make_async_copy(outv0, dummy, sem0).wait()
    pltpu.make_async_copy(outv1, dummy, sem1).wait()


@jax.jit
def kernel(attn_bias, spatial_pos, W_spatial, W_spatial_rev):
    sp = spatial_pos.astype(jnp.int32)

    def pack_pairs(w):
        u = jax.lax.bitcast_convert_type(
            w.astype(jnp.bfloat16), jnp.uint16).astype(jnp.uint32)
        packed = u[:, 0::2] | (u[:, 1::2] << 16)
        return jax.lax.bitcast_convert_type(
            jnp.transpose(packed), jnp.int32).reshape(-1)

    wt = pack_pairs(W_spatial)
    wrt = pack_pairs(W_spatial_rev)
    run = pl.kernel(
        _body,
        out_type=jax.ShapeDtypeStruct((B, H, N, N), jnp.float32),
        mesh=plsc.VectorSubcoreMesh(core_axis_name="c", subcore_axis_name="s"),
        compiler_params=pltpu.CompilerParams(needs_layout_passes=False,
                                             use_tc_tiling_on_sc=True),
        scratch_types=[
            pltpu.VMEM((BK, BK), jnp.int32),
            pltpu.VMEM((BK, BK), jnp.int32),
            pltpu.VMEM((BK * PT,), jnp.int32),
            pltpu.VMEM((BK, BK), jnp.float32),
            pltpu.VMEM((H // 2 * S,), jnp.int32),
            pltpu.VMEM((H // 2 * S,), jnp.int32),
            pltpu.VMEM((H, ISUB, BK), jnp.float32),
            pltpu.VMEM((H, ISUB, BK), jnp.float32),
            pltpu.SemaphoreType.DMA,
            pltpu.SemaphoreType.DMA,
        ],
    )
    return run(attn_bias, sp, wt, wrt)


# speedup vs baseline: 1.0863x; 1.0863x over previous
import jax
import jax.numpy as jnp
from jax import lax
from jax.experimental import pallas as pl
from jax.experimental.pallas import tpu as pltpu
from jax.experimental.pallas import tpu_sc as plsc

B = 8
N = 512
H = 16
S = 512
L = 16
NC = 2
NS = 16
NW = NC * NS
BK = 128
PT = 136
NB = N // BK
TOT = B * NB * NB
PER = TOT // NW
ISUB = 8
JV = BK // L


def _decode(t):
    b = t // (NB * NB)
    r = t % (NB * NB)
    return b, (r // NB) * BK, (r % NB) * BK


def _body(ab_hbm, sp_hbm, wt_hbm, wrt_hbm, out_hbm,
          spA0, spA1, spB, spBT, abA, wv, wrv, outv0, outv1,
          sem0, sem1, semA, semB):
    c = lax.axis_index("c")
    s = lax.axis_index("s")
    wid = s * NC + c
    pltpu.sync_copy(wt_hbm, wv)
    pltpu.sync_copy(wrt_hbm, wrv)
    lane = lax.iota(jnp.int32, L)
    lane_pt = lane * PT

    dummy = out_hbm.at[0, :, pl.ds(0, ISUB), pl.ds(0, BK)]
    pltpu.async_copy(dummy, outv0, sem0)
    pltpu.async_copy(dummy, outv1, sem1)

    t0 = wid * PER
    b0, i00, j00 = _decode(t0)
    pltpu.async_copy(sp_hbm.at[b0, pl.ds(i00, BK), pl.ds(j00, BK)], spA0,
                     semA)
    pltpu.async_copy(sp_hbm.at[b0, pl.ds(j00, BK), pl.ds(i00, BK)], spB,
                     semB)

    def fill(outv, spA, isub):
        @plsc.parallel_loop(0, ISUB * JV, unroll=1)
        def pix_body(p):
            i2 = p // JV
            jv = p % JV
            i = isub * ISUB + i2
            v_idx = spA[i, pl.ds(jv * L, L)]
            vt_idx = spBT[pl.ds(i * PT + jv * L, L)]
            ab_v = abA[i, pl.ds(jv * L, L)]
            for hp in range(H // 2):
                g = plsc.load_gather(wv, [v_idx + hp * S])
                gr = plsc.load_gather(wrv, [vt_idx + hp * S])
                ssum = (plsc.bitcast(g, jnp.bfloat16)
                        + plsc.bitcast(gr, jnp.bfloat16))
                lo, hi = plsc.unpack(ssum, format=plsc.PackFormat.INTERLEAVED,
                                     preferred_element_type=jnp.float32)
                outv[2 * hp, i2, pl.ds(jv * L, L)] = lo + ab_v
                outv[2 * hp + 1, i2, pl.ds(jv * L, L)] = hi + ab_v

    def do_block(t, spA_cur, spA_nxt):
        b, i0, j0 = _decode(t)
        tn = jnp.minimum(t + 1, TOT - 1)
        bn, i0n, j0n = _decode(tn)
        pltpu.make_async_copy(
            sp_hbm.at[0, pl.ds(0, BK), pl.ds(0, BK)], spA_cur, semA).wait()
        pltpu.make_async_copy(
            sp_hbm.at[0, pl.ds(0, BK), pl.ds(0, BK)], spB, semB).wait()
        pltpu.sync_copy(ab_hbm.at[b, pl.ds(i0, BK), pl.ds(j0, BK)], abA)
        pltpu.async_copy(sp_hbm.at[bn, pl.ds(i0n, BK), pl.ds(j0n, BK)],
                         spA_nxt, semA)

        @plsc.parallel_loop(0, BK * JV, unroll=1)
        def tr_body(g):
            j = g // JV
            iv = g % JV
            row = spB[j, pl.ds(iv * L, L)]
            plsc.store_scatter(spBT, [lane_pt + (iv * L * PT + j)], row)

        pltpu.async_copy(sp_hbm.at[bn, pl.ds(j0n, BK), pl.ds(i0n, BK)], spB,
                         semB)

        def isub2_body(k2, _):
            for half, (ov, sem) in enumerate(((outv0, sem0), (outv1, sem1))):
                isub = k2 * 2 + half
                dst = out_hbm.at[b, :, pl.ds(i0 + isub * ISUB, ISUB),
                                 pl.ds(j0, BK)]
                pltpu.make_async_copy(ov, dst, sem).wait()
                fill(ov, spA_cur, isub)
                pltpu.async_copy(ov, dst, sem)
            return _

        lax.fori_loop(0, (BK // ISUB) // 2, isub2_body, 0)

    def pair_body(kk, carry):
        t = wid * PER + 2 * kk
        do_block(t, spA0, spA1)
        do_block(t + 1, spA1, spA0)
        return carry

    lax.fori_loop(0, PER // 2, pair_body, 0)
    pltpu.make_async_copy(outv0, dummy, sem0).wait()
    pltpu.make_async_copy(outv1, dummy, sem1).wait()
    pltpu.make_async_copy(
        sp_hbm.at[0, pl.ds(0, BK), pl.ds(0, BK)], spA0, semA).wait()
    pltpu.make_async_copy(
        sp_hbm.at[0, pl.ds(0, BK), pl.ds(0, BK)], spB, semB).wait()


@jax.jit
def kernel(attn_bias, spatial_pos, W_spatial, W_spatial_rev):
    sp = spatial_pos.astype(jnp.int32)

    def pack_pairs(w):
        u = jax.lax.bitcast_convert_type(
            w.astype(jnp.bfloat16), jnp.uint16).astype(jnp.uint32)
        packed = u[:, 0::2] | (u[:, 1::2] << 16)
        return jax.lax.bitcast_convert_type(
            jnp.transpose(packed), jnp.int32).reshape(-1)

    wt = pack_pairs(W_spatial)
    wrt = pack_pairs(W_spatial_rev)
    run = pl.kernel(
        _body,
        out_type=jax.ShapeDtypeStruct((B, H, N, N), jnp.float32),
        mesh=plsc.VectorSubcoreMesh(core_axis_name="c", subcore_axis_name="s"),
        compiler_params=pltpu.CompilerParams(needs_layout_passes=False,
                                             use_tc_tiling_on_sc=True),
        scratch_types=[
            pltpu.VMEM((BK, BK), jnp.int32),
            pltpu.VMEM((BK, BK), jnp.int32),
            pltpu.VMEM((BK, BK), jnp.int32),
            pltpu.VMEM((BK * PT,), jnp.int32),
            pltpu.VMEM((BK, BK), jnp.float32),
            pltpu.VMEM((H // 2 * S,), jnp.int32),
            pltpu.VMEM((H // 2 * S,), jnp.int32),
            pltpu.VMEM((H, ISUB, BK), jnp.float32),
            pltpu.VMEM((H, ISUB, BK), jnp.float32),
            pltpu.SemaphoreType.DMA,
            pltpu.SemaphoreType.DMA,
            pltpu.SemaphoreType.DMA,
            pltpu.SemaphoreType.DMA,
        ],
    )
    return run(attn_bias, sp, wt, wrt)
